# baseline (speedup 1.0000x reference)
import functools

import jax
import jax.numpy as jnp
from jax import lax
from jax.experimental import pallas as pl
from jax.experimental.pallas import tpu as pltpu
from jax.experimental.pallas import tpu_sc as plsc

_B = 16384
_D = 32
_NSC = 21
_NC, _NS = 2, 16
_NW = _NC * _NS
_ROWS_W = _B // _NW
_CHUNK = 16
_NCHUNK = _ROWS_W // _CHUNK
_CTX_PER_CHUNK = _CHUNK * _NSC
_GSPLIT = 3
_GLEN = _CTX_PER_CHUNK // _GSPLIT
_M = _B * _NSC


def _sc_partial_body(cW_hbm, xW_hbm, cidx_hbm, xidx_hbm, out_hbm,
                     cidx_v, xidx_v, crow_v, xrow_v, part_v, sem):
    wid = lax.axis_index("s") * _NC + lax.axis_index("c")
    base = wid * _ROWS_W
    pltpu.sync_copy(cidx_hbm.at[pl.ds(base, _ROWS_W)], cidx_v)
    pltpu.sync_copy(xidx_hbm.at[pl.ds(base * _NSC, _ROWS_W * _NSC)], xidx_v)

    def chunk_body(ci, carry):
        rbase = ci * _CHUNK
        dmas = [pltpu.async_copy(
            cW_hbm.at[cidx_v.at[pl.ds(rbase, _CHUNK)]], crow_v, sem)]
        for g in range(_GSPLIT):
            dmas.append(pltpu.async_copy(
                xW_hbm.at[xidx_v.at[pl.ds(rbase * _NSC + g * _GLEN, _GLEN)]],
                xrow_v.at[pl.ds(g * _GLEN, _GLEN)], sem))
        for d in dmas:
            d.wait()

        def row_body(r, rcarry):
            c_lo = crow_v[r, pl.ds(0, 16)]
            c_hi = crow_v[r, pl.ds(16, 16)]
            ncl, nch = -c_lo, -c_hi
            for j in range(_NSC):
                x_lo = xrow_v[r * _NSC + j, pl.ds(0, 16)]
                x_hi = xrow_v[r * _NSC + j, pl.ds(16, 16)]
                if j == 0:
                    part_v[r * _NSC + j, :] = c_lo * x_lo + c_hi * x_hi
                else:
                    part_v[r * _NSC + j, :] = ncl * x_lo + nch * x_hi
            return rcarry

        lax.fori_loop(0, _CHUNK, row_body, 0)
        pltpu.sync_copy(part_v, out_hbm.at[wid * _NCHUNK + ci])
        return carry

    lax.fori_loop(0, _NCHUNK, chunk_body, 0)


_sc_partial = functools.partial(
    pl.kernel,
    mesh=plsc.VectorSubcoreMesh(core_axis_name="c", subcore_axis_name="s"),
    out_type=jax.ShapeDtypeStruct((_B // _CHUNK, _CTX_PER_CHUNK, 16),
                                  jnp.float32),
    scratch_types=[
        pltpu.VMEM((_ROWS_W,), jnp.int32),
        pltpu.VMEM((_ROWS_W * _NSC,), jnp.int32),
        pltpu.VMEM((_CHUNK, _D), jnp.float32),
        pltpu.VMEM((_CTX_PER_CHUNK, _D), jnp.float32),
        pltpu.VMEM((_CTX_PER_CHUNK, 16), jnp.float32),
        pltpu.SemaphoreType.DMA,
    ],
    compiler_params=pltpu.CompilerParams(use_tc_tiling_on_sc=False),
)(_sc_partial_body)

_BLK = 8192
_NBLK = _M // _BLK


def _loss_body(p_ref, o_ref):
    i = pl.program_id(0)
    x = jnp.sum(p_ref[...], axis=1)
    ls = jnp.minimum(x, 0.0) - jnp.log1p(jnp.exp(-jnp.abs(x)))

    @pl.when(i == 0)
    def _init():
        o_ref[...] = jnp.zeros((1, 1), jnp.float32)

    o_ref[...] += jnp.sum(ls).reshape(1, 1)

    @pl.when(i == _NBLK - 1)
    def _fini():
        o_ref[...] = -o_ref[...] / _B


_loss_call = pl.pallas_call(
    _loss_body,
    grid=(_NBLK,),
    in_specs=[pl.BlockSpec((_BLK, 16), lambda i: (i, 0))],
    out_specs=pl.BlockSpec((1, 1), lambda i: (0, 0)),
    out_shape=jax.ShapeDtypeStruct((1, 1), jnp.float32),
)


def kernel(center, context, negatives, center_W, context_W):
    cidx = center.reshape(_B).astype(jnp.int32)
    xidx = jnp.concatenate(
        [context.astype(jnp.int32), negatives.astype(jnp.int32)],
        axis=1).reshape(_B * _NSC)
    part = _sc_partial(center_W, context_W, cidx, xidx)
    return _loss_call(part.reshape(_M, 16)).reshape(())


# speedup vs baseline: 1.1919x; 1.1919x over previous
import functools

import jax
import jax.numpy as jnp
from jax import lax
from jax.experimental import pallas as pl
from jax.experimental.pallas import tpu as pltpu
from jax.experimental.pallas import tpu_sc as plsc

_B = 16384
_D = 32
_NNEG = 20
_NSC = 21
_NC, _NS = 2, 16
_NW = _NC * _NS
_ROWS_W = _B // _NW
_CHUNK = 16
_NCHUNK = _ROWS_W // _CHUNK
_NEG_PER_CHUNK = _CHUNK * _NNEG
_VEC_PER_CHUNK = _CHUNK * _NSC
_OUT_ROWS_PER_CHUNK = _VEC_PER_CHUNK * 16 // 128
_OUT_ROWS = _B * _NSC * 16 // 128


def _sc_partial_body(cW_hbm, xW_hbm, cidx_hbm, pidx_hbm, nidx_hbm, out_hbm,
                     cidx_v, pidx_v, nidx_v, crow_v, prow_v, nrow_v,
                     part_v, sem):
    wid = lax.axis_index("s") * _NC + lax.axis_index("c")
    base = wid * _ROWS_W
    pltpu.sync_copy(cidx_hbm.at[pl.ds(base, _ROWS_W)], cidx_v)
    pltpu.sync_copy(pidx_hbm.at[pl.ds(base, _ROWS_W)], pidx_v)
    pltpu.sync_copy(nidx_hbm.at[pl.ds(base * _NNEG, _ROWS_W * _NNEG)], nidx_v)

    def chunk_body(ci, carry):
        rbase = ci * _CHUNK
        nbase = ci * _NEG_PER_CHUNK
        dmas = [
            pltpu.async_copy(
                cW_hbm.at[cidx_v.at[pl.ds(rbase, _CHUNK)]], crow_v, sem),
            pltpu.async_copy(
                xW_hbm.at[pidx_v.at[pl.ds(rbase, _CHUNK)]], prow_v, sem),
        ]
        for g, (off, ln) in enumerate(((0, 128), (128, 128), (256, 64))):
            dmas.append(pltpu.async_copy(
                xW_hbm.at[nidx_v.at[pl.ds(nbase + off, ln)]],
                nrow_v.at[pl.ds(off, ln)], sem))
        for d in dmas:
            d.wait()

        def row_body(r, rcarry):
            c_lo = crow_v[r, pl.ds(0, 16)]
            c_hi = crow_v[r, pl.ds(16, 16)]
            ncl, nch = -c_lo, -c_hi
            x_lo = prow_v[r, pl.ds(0, 16)]
            x_hi = prow_v[r, pl.ds(16, 16)]
            part_v[r // 8, pl.ds((r % 8) * 16, 16)] = c_lo * x_lo + c_hi * x_hi
            for j in range(_NNEG):
                k = r * _NNEG + j
                n_lo = nrow_v[k, pl.ds(0, 16)]
                n_hi = nrow_v[k, pl.ds(16, 16)]
                part_v[2 + k // 8, pl.ds((k % 8) * 16, 16)] = (
                    ncl * n_lo + nch * n_hi)
            return rcarry

        lax.fori_loop(0, _CHUNK, row_body, 0)
        pltpu.sync_copy(
            part_v,
            out_hbm.at[pl.ds((wid * _NCHUNK + ci) * _OUT_ROWS_PER_CHUNK,
                             _OUT_ROWS_PER_CHUNK)])
        return carry

    lax.fori_loop(0, _NCHUNK, chunk_body, 0)


_sc_partial = functools.partial(
    pl.kernel,
    mesh=plsc.VectorSubcoreMesh(core_axis_name="c", subcore_axis_name="s"),
    out_type=jax.ShapeDtypeStruct((_OUT_ROWS, 128), jnp.float32),
    scratch_types=[
        pltpu.VMEM((_ROWS_W,), jnp.int32),
        pltpu.VMEM((_ROWS_W,), jnp.int32),
        pltpu.VMEM((_ROWS_W * _NNEG,), jnp.int32),
        pltpu.VMEM((_CHUNK, _D), jnp.float32),
        pltpu.VMEM((_CHUNK, _D), jnp.float32),
        pltpu.VMEM((_NEG_PER_CHUNK, _D), jnp.float32),
        pltpu.VMEM((_OUT_ROWS_PER_CHUNK, 128), jnp.float32),
        pltpu.SemaphoreType.DMA,
    ],
    compiler_params=pltpu.CompilerParams(use_tc_tiling_on_sc=False),
)(_sc_partial_body)

_BLK = 7168
_NBLK = _OUT_ROWS // _BLK


def _loss_body(p_ref, o_ref):
    i = pl.program_id(0)
    x = p_ref[...]
    lane = lax.broadcasted_iota(jnp.int32, (128, 8), 0)
    grp = lax.broadcasted_iota(jnp.int32, (128, 8), 1)
    m = jnp.where(lane // 16 == grp, 1.0, 0.0).astype(jnp.float32)
    s = jnp.dot(x, m, preferred_element_type=jnp.float32)
    ls = jnp.minimum(s, 0.0) - jnp.log1p(jnp.exp(-jnp.abs(s)))

    @pl.when(i == 0)
    def _init():
        o_ref[...] = jnp.zeros((1, 1), jnp.float32)

    o_ref[...] += jnp.sum(ls).reshape(1, 1)

    @pl.when(i == _NBLK - 1)
    def _fini():
        o_ref[...] = -o_ref[...] / _B


_loss_call = pl.pallas_call(
    _loss_body,
    grid=(_NBLK,),
    in_specs=[pl.BlockSpec((_BLK, 128), lambda i: (i, 0))],
    out_specs=pl.BlockSpec((1, 1), lambda i: (0, 0)),
    out_shape=jax.ShapeDtypeStruct((1, 1), jnp.float32),
)


def kernel(center, context, negatives, center_W, context_W):
    cidx = center.reshape(_B).astype(jnp.int32)
    pidx = context.reshape(_B).astype(jnp.int32)
    nidx = negatives.reshape(_B * _NNEG).astype(jnp.int32)
    part = _sc_partial(center_W, context_W, cidx, pidx, nidx)
    return _loss_call(part).reshape(())
